# Initial kernel scaffold; baseline (speedup 1.0000x reference)
import jax
import jax.numpy as jnp
from jax.experimental import pallas as pl


def kernel(x, ei, W1, b1, W2, b2):
    raise NotImplementedError("write your pallas kernel here")


# speedup vs baseline: 1.0264x; 1.0264x over previous
import jax
import jax.numpy as jnp
from jax import lax
from jax.experimental import pallas as pl

_N = 10000
_K = 10
_ALPHA = 0.1


def _mlp_body(x_ref, w1_ref, b1_ref, w2_ref, b2_ref, o_ref):
    x = x_ref[...]
    h = lax.dot_general(x, w1_ref[...], (((1,), (1,)), ((), ())),
                        preferred_element_type=jnp.float32)
    h = jnp.maximum(h + b1_ref[...], 0.0)
    o = lax.dot_general(h, w2_ref[...], (((1,), (1,)), ((), ())),
                        preferred_element_type=jnp.float32)
    o_ref[...] = o + b2_ref[...]


def _mlp(x, W1, b1, W2, b2):
    blk = 2000
    return pl.pallas_call(
        _mlp_body,
        grid=(_N // blk,),
        in_specs=[
            pl.BlockSpec((blk, 128), lambda i: (i, 0)),
            pl.BlockSpec((128, 128), lambda i: (0, 0)),
            pl.BlockSpec((128,), lambda i: (0,)),
            pl.BlockSpec((128, 128), lambda i: (0, 0)),
            pl.BlockSpec((128,), lambda i: (0,)),
        ],
        out_specs=pl.BlockSpec((blk, 128), lambda i: (i, 0)),
        out_shape=jax.ShapeDtypeStruct((_N, 128), jnp.float32),
    )(x, W1, b1, W2, b2)


def kernel(x, ei, W1, b1, W2, b2):
    h = _mlp(x, W1, b1, W2, b2)

    loop = jnp.arange(_N, dtype=ei.dtype)
    row = jnp.concatenate([ei[0], loop])
    col = jnp.concatenate([ei[1], loop])
    ew = jnp.ones(row.shape[0], dtype=jnp.float32)
    deg = jax.ops.segment_sum(ew, col, num_segments=_N)
    deg_inv_sqrt = jnp.where(deg > 0, 1.0 / jnp.sqrt(deg), 0.0)
    norm = deg_inv_sqrt[row] * ew * deg_inv_sqrt[col]

    z = h
    for _ in range(_K):
        msg = norm[:, None] * z[row]
        agg = jax.ops.segment_sum(msg, col, num_segments=_N)
        z = (1.0 - _ALPHA) * agg + _ALPHA * h
    return z


# speedup vs baseline: 4.5168x; 4.4007x over previous
import jax
import jax.numpy as jnp
from jax import lax
from jax.experimental import pallas as pl
from jax.experimental.pallas import tpu as pltpu
from jax.experimental.pallas import tpu_sc as plsc

_N = 10000
_E = 320000
_D = 128
_K = 10
_ALPHA = 0.1

_NC = 2
_NS = 16
_NW = _NC * _NS
_NPT = 320
_NPAD = _NW * _NPT
_CHUNK = 128
_PAIR = 2 * _CHUNK
_ECHUNK = 4000
_NSCAN = _E // _ECHUNK
_LBUF = 4608
_BLK = 512
_CAP = _E + 512


def _mlp_body(x_ref, w1_ref, b1_ref, w2_ref, b2_ref, o_ref):
    x = x_ref[...]
    h = lax.dot_general(x, w1_ref[...], (((1,), (1,)), ((), ())),
                        preferred_element_type=jnp.float32)
    h = jnp.maximum(h + b1_ref[...], 0.0)
    o = lax.dot_general(h, w2_ref[...], (((1,), (1,)), ((), ())),
                        preferred_element_type=jnp.float32)
    o_ref[...] = o + b2_ref[...]


def _mlp(x, W1, b1, W2, b2):
    blk = 2000
    return pl.pallas_call(
        _mlp_body,
        grid=(_N // blk,),
        in_specs=[
            pl.BlockSpec((blk, _D), lambda i: (i, 0)),
            pl.BlockSpec((_D, _D), lambda i: (0, 0)),
            pl.BlockSpec((_D,), lambda i: (0,)),
            pl.BlockSpec((_D, _D), lambda i: (0, 0)),
            pl.BlockSpec((_D,), lambda i: (0,)),
        ],
        out_specs=pl.BlockSpec((blk, _D), lambda i: (i, 0)),
        out_shape=jax.ShapeDtypeStruct((_N, _D), jnp.float32),
    )(x, W1, b1, W2, b2)


def _m8(i):
    return pl.multiple_of(i, 8)


def _mesh():
    return plsc.VectorSubcoreMesh(
        core_axis_name="c", subcore_axis_name="s",
        num_cores=_NC, num_subcores=_NS)


_SC_PARAMS = pltpu.CompilerParams(needs_layout_passes=False)


def _pre_body(rows_hbm, cols_hbm, h_hbm,
              lrow_hbm, lcol_hbm, cnt_hbm, dis_hbm, zp_hbm,
              rbuf, cbuf, lrow_v, lcol_v, hbuf, degb, colv, cntv, sem):
    cid = lax.axis_index("c")
    sid = lax.axis_index("s")
    wid = cid * _NS + sid
    base = wid * _NPT
    lo = base
    hi = base + _NPT
    lbase = wid * _CAP

    lo16 = jnp.full((16,), lo, jnp.int32)
    hi16 = jnp.full((16,), hi, jnp.int32)

    def scan_chunk(i, carry):
        cur, gcur = carry
        off = i * _ECHUNK
        pltpu.sync_copy(rows_hbm.at[pl.ds(off, _ECHUNK)], rbuf)
        pltpu.sync_copy(cols_hbm.at[pl.ds(off, _ECHUNK)], cbuf)

        def group(g, cur):
            c16 = cbuf[pl.ds(g * 16, 16)]
            r16 = rbuf[pl.ds(g * 16, 16)]
            m = (c16 >= lo16) & (c16 < hi16)
            mi = m.astype(jnp.int32)
            incl = plsc.cumsum(mi)
            idx = (jnp.full((16,), cur, jnp.int32) + incl) - mi
            plsc.store_scatter(lrow_v, [idx], r16, mask=m)
            plsc.store_scatter(lcol_v, [idx], c16 - lo16, mask=m)
            return cur + incl[15]

        cur = lax.fori_loop(0, _ECHUNK // 16, group, cur)

        nblk = cur // _BLK

        def flush(b, _):
            pltpu.sync_copy(lrow_v.at[pl.ds(b * _BLK, _BLK)],
                            lrow_hbm.at[pl.ds(_m8(lbase + gcur + b * _BLK), _BLK)])
            pltpu.sync_copy(lcol_v.at[pl.ds(b * _BLK, _BLK)],
                            lcol_hbm.at[pl.ds(_m8(lbase + gcur + b * _BLK), _BLK)])
            return 0

        lax.fori_loop(0, nblk, flush, 0)

        @pl.when(nblk > 0)
        def _move_remainder():
            def mv(g, _):
                lrow_v[pl.ds(g * 16, 16)] = lrow_v[pl.ds(nblk * _BLK + g * 16, 16)]
                lcol_v[pl.ds(g * 16, 16)] = lcol_v[pl.ds(nblk * _BLK + g * 16, 16)]
                return 0
            lax.fori_loop(0, _BLK // 16, mv, 0)

        return cur - nblk * _BLK, gcur + nblk * _BLK

    cur, gcur = lax.fori_loop(0, _NSCAN, scan_chunk,
                              (jnp.int32(0), jnp.int32(0)))

    total = gcur + cur
    padded = ((total + _PAIR + _PAIR - 1) // _PAIR) * _PAIR
    iota = lax.iota(jnp.int32, 16)
    base16 = jnp.full((16,), base, jnp.int32)

    def padg(g, _):
        off = cur + g * 16
        g16 = jnp.full((16,), g * 16, jnp.int32)
        lrow_v[pl.ds(off, 16)] = base16 + ((g16 + iota) & 255)
        lcol_v[pl.ds(off, 16)] = jnp.full((16,), _NPT, jnp.int32)
        return 0

    lax.fori_loop(0, 2 * _PAIR // 16, padg, 0)

    nb8 = (padded - gcur) // 8

    def flush8(b, _):
        pltpu.sync_copy(lrow_v.at[pl.ds(b * 8, 8)],
                        lrow_hbm.at[pl.ds(_m8(lbase + gcur + b * 8), 8)])
        pltpu.sync_copy(lcol_v.at[pl.ds(b * 8, 8)],
                        lcol_hbm.at[pl.ds(_m8(lbase + gcur + b * 8), 8)])
        return 0

    lax.fori_loop(0, nb8, flush8, 0)

    cntv[pl.ds(0, 16)] = jnp.full((16,), padded, jnp.int32)
    pltpu.sync_copy(cntv.at[pl.ds(0, 8)], cnt_hbm.at[pl.ds(_m8(wid * 8), 8)])

    ones16 = jnp.full((16,), 1.0, jnp.float32)
    lane0 = jnp.where(iota == jnp.full((16,), 0, jnp.int32),
                      jnp.full((16,), 1.0, jnp.float32),
                      jnp.full((16,), 0.0, jnp.float32))

    def dinit(g, _):
        degb[pl.ds(g * 16, 16)] = ones16
        return 0

    lax.fori_loop(0, (_NPT + 32) // 16, dinit, 0)

    def degchunk(k, _):
        pltpu.sync_copy(lcol_hbm.at[pl.ds(_m8(lbase + k * _CHUNK), _CHUNK)],
                        colv.at[pl.ds(0, _CHUNK)])

        def one(e, _):
            c = colv[pl.ds(e, 16)][0]
            degb[pl.ds(c, 16)] = degb[pl.ds(c, 16)] + lane0
            return 0

        lax.fori_loop(0, _CHUNK, one, 0)
        return 0

    lax.fori_loop(0, padded // _CHUNK, degchunk, 0)

    def rsq(g, _):
        d = degb[pl.ds(g * 16, 16)]
        ibits = plsc.bitcast(d, jnp.int32)
        ibits = jnp.full((16,), 0x5F3759DF, jnp.int32) - (ibits >> 1)
        y = plsc.bitcast(ibits, jnp.float32)
        for _unused in range(3):
            y = y * (1.5 - 0.5 * d * y * y)
        degb[pl.ds(g * 16, 16)] = y
        return 0

    lax.fori_loop(0, _NPT // 16, rsq, 0)
    pltpu.sync_copy(degb.at[pl.ds(0, _NPT)], dis_hbm.at[pl.ds(_m8(base), _NPT)])

    pltpu.sync_copy(h_hbm.at[pl.ds(base, _NPT)], hbuf)

    def zrow(c, _):
        t16 = jnp.full((16,), degb[pl.ds(c, 16)][0], jnp.float32)
        for j in range(8):
            hbuf[c, pl.ds(j * 16, 16)] = hbuf[c, pl.ds(j * 16, 16)] * t16
        return 0

    lax.fori_loop(0, _NPT, zrow, 0)
    pltpu.sync_copy(hbuf, zp_hbm.at[pl.ds(base, _NPT)])


_pre = pl.kernel(
    _pre_body,
    out_type=(
        jax.ShapeDtypeStruct((_NW * _CAP,), jnp.int32),
        jax.ShapeDtypeStruct((_NW * _CAP,), jnp.int32),
        jax.ShapeDtypeStruct((_NW * 8,), jnp.int32),
        jax.ShapeDtypeStruct((_NW * _NPT,), jnp.float32),
        jax.ShapeDtypeStruct((_NPAD, _D), jnp.float32),
    ),
    mesh=_mesh(),
    compiler_params=_SC_PARAMS,
    scratch_types=[
        pltpu.VMEM((_ECHUNK,), jnp.int32),
        pltpu.VMEM((_ECHUNK,), jnp.int32),
        pltpu.VMEM((_LBUF,), jnp.int32),
        pltpu.VMEM((_LBUF,), jnp.int32),
        pltpu.VMEM((_NPT, _D), jnp.float32),
        pltpu.VMEM((_NPT + 32,), jnp.float32),
        pltpu.VMEM((_CHUNK + 16,), jnp.int32),
        pltpu.VMEM((16,), jnp.int32),
        pltpu.SemaphoreType.DMA,
    ],
)


def _prop_body(lrow_hbm, lcol_hbm, cnt_hbm, dis_hbm, h_hbm, zp_hbm,
               zpo_hbm, zo_hbm,
               acc, hbuf, gbuf0, gbuf1, idx0, idx1,
               colv, disv, cntv, sem0, sem1):
    cid = lax.axis_index("c")
    sid = lax.axis_index("s")
    wid = cid * _NS + sid
    base = wid * _NPT
    lbase = wid * _CAP

    pltpu.sync_copy(cnt_hbm.at[pl.ds(_m8(wid * 8), 8)], cntv.at[pl.ds(0, 8)])
    pltpu.sync_copy(dis_hbm.at[pl.ds(_m8(base), _NPT)], disv.at[pl.ds(0, _NPT)])
    pltpu.sync_copy(zp_hbm.at[pl.ds(base, _NPT)], acc.at[pl.ds(0, _NPT)])
    pltpu.sync_copy(h_hbm.at[pl.ds(base, _NPT)], hbuf)

    cnt = cntv[pl.ds(0, 16)][0]
    npair = cnt // _PAIR

    def process(gbuf, coff):
        pltpu.sync_copy(lcol_hbm.at[pl.ds(_m8(lbase + coff), _CHUNK)],
                        colv.at[pl.ds(0, _CHUNK)])

        def one(e, _):
            c = colv[pl.ds(e, 16)][0]
            for j in range(8):
                plsc.addupdate(acc.at[c, pl.ds(j * 16, 16)],
                               gbuf[e, pl.ds(j * 16, 16)])
            return 0

        lax.fori_loop(0, _CHUNK, one, 0)

    def pair(p, _):
        o0 = p * _PAIR
        o1 = o0 + _CHUNK
        pltpu.sync_copy(lrow_hbm.at[pl.ds(_m8(lbase + o0), _CHUNK)], idx0)
        cp0 = pltpu.async_copy(zp_hbm.at[idx0], gbuf0, sem0)
        pltpu.sync_copy(lrow_hbm.at[pl.ds(_m8(lbase + o1), _CHUNK)], idx1)
        cp1 = pltpu.async_copy(zp_hbm.at[idx1], gbuf1, sem1)
        cp0.wait()
        process(gbuf0, o0)
        cp1.wait()
        process(gbuf1, o1)
        return 0

    lax.fori_loop(0, npair, pair, 0)

    def fin(c, _):
        t = disv[pl.ds(c, 16)][0]
        t16 = jnp.full((16,), t, jnp.float32)
        a16 = jnp.full((16,), (1.0 - _ALPHA) * t, jnp.float32)
        for j in range(8):
            s = acc[c, pl.ds(j * 16, 16)]
            hv = hbuf[c, pl.ds(j * 16, 16)]
            z = a16 * s + _ALPHA * hv
            hbuf[c, pl.ds(j * 16, 16)] = z
            acc[c, pl.ds(j * 16, 16)] = t16 * z
        return 0

    lax.fori_loop(0, _NPT, fin, 0)
    pltpu.sync_copy(hbuf, zo_hbm.at[pl.ds(base, _NPT)])
    pltpu.sync_copy(acc.at[pl.ds(0, _NPT)], zpo_hbm.at[pl.ds(base, _NPT)])


_prop = pl.kernel(
    _prop_body,
    out_type=(
        jax.ShapeDtypeStruct((_NPAD, _D), jnp.float32),
        jax.ShapeDtypeStruct((_NPAD, _D), jnp.float32),
    ),
    mesh=_mesh(),
    compiler_params=_SC_PARAMS,
    scratch_types=[
        pltpu.VMEM((_NPT + 1, _D), jnp.float32),
        pltpu.VMEM((_NPT, _D), jnp.float32),
        pltpu.VMEM((_CHUNK, _D), jnp.float32),
        pltpu.VMEM((_CHUNK, _D), jnp.float32),
        pltpu.VMEM((_CHUNK,), jnp.int32),
        pltpu.VMEM((_CHUNK,), jnp.int32),
        pltpu.VMEM((_CHUNK + 16,), jnp.int32),
        pltpu.VMEM((_NPT + 16,), jnp.float32),
        pltpu.VMEM((16,), jnp.int32),
        pltpu.SemaphoreType.DMA,
        pltpu.SemaphoreType.DMA,
    ],
)


def kernel(x, ei, W1, b1, W2, b2):
    h = _mlp(x, W1, b1, W2, b2)
    hp = jnp.pad(h, ((0, _NPAD - _N), (0, 0)))
    rows = ei[0].astype(jnp.int32)
    cols = ei[1].astype(jnp.int32)
    lrow, lcol, cnt, dis, zp = _pre(rows, cols, hp)
    z = hp
    for _ in range(_K):
        zp, z = _prop(lrow, lcol, cnt, dis, hp, zp)
    return z[:_N]
